# baseline (speedup 1.0000x reference)
import functools

import jax
import jax.numpy as jnp
from jax import lax
from jax.experimental import pallas as pl
from jax.experimental.pallas import tpu as pltpu
from jax.experimental.pallas import tpu_sc as plsc

_RES = 1024
_MIN_B = -2000.0
_MAX_B = 3000.0
_SCALE = (_RES - 1) / (_MAX_B - _MIN_B)

_MAGIC = 12582912.0
_CLO = _MAGIC
_CHI = _MAGIC + (_RES - 1)

_NC = 2
_NS = 16
_NW = _NC * _NS
_B, _N = 64, 4096
_TOTAL = _B * _N
_PPW = _TOTAL // _NW
_ROW = 128
_ROWS_PW = _PPW // _ROW
_VPR = _ROW // 16


def _quant(v):
  t = (v + jnp.float32(-_MIN_B)) * jnp.float32(_SCALE)
  z = t + jnp.float32(_MAGIC)
  z = jnp.minimum(jnp.maximum(z, jnp.float32(_CLO)), jnp.float32(_CHI))
  return plsc.bitcast(z, jnp.int32) & (_RES - 1)


_WSLICE = _RES * _RES // _NS


def _sc_kernel(x_hbm, w_hbm, out_hbm, wsh, xv, idxv, outv, sem, wsem):
  sid = lax.axis_index("s")
  wid = sid * _NC + lax.axis_index("c")
  pltpu.make_async_copy(
      w_hbm.at[pl.ds(sid * _WSLICE, _WSLICE)],
      wsh.at[pl.ds(sid * _WSLICE, _WSLICE)],
      wsem,
  ).start()
  pltpu.sync_copy(x_hbm.at[pl.ds(wid * (2 * _PPW), 2 * _PPW)], xv)

  iota2 = lax.iota(jnp.int32, 16) * 2

  def row(j, carry):
    for t in range(_VPR):
      base = j * (2 * _ROW) + t * 32
      ib = iota2 + base
      b = plsc.load_gather(xv, [ib])
      d = plsc.load_gather(xv, [ib + 1])
      flat = (_quant(b) << 10) | _quant(d)
      idxv[j, pl.ds(t * 16, 16)] = flat
    return carry

  lax.fori_loop(0, _ROWS_PW, row, 0)

  plsc.subcore_barrier()

  def fire(j, carry):
    pltpu.make_async_copy(wsh.at[idxv.at[j]], outv.at[j], sem).start()
    return carry

  lax.fori_loop(0, _ROWS_PW, fire, 0)

  def drain(j, carry):
    pltpu.make_async_copy(wsh.at[idxv.at[j]], outv.at[j], sem).wait()
    return carry

  lax.fori_loop(0, _ROWS_PW, drain, 0)
  pltpu.sync_copy(outv, out_hbm.at[pl.ds(wid * _ROWS_PW, _ROWS_PW)])


@jax.jit
def kernel(x, w):
  mesh = plsc.VectorSubcoreMesh(core_axis_name="c", subcore_axis_name="s")
  run = functools.partial(
      pl.kernel,
      mesh=mesh,
      compiler_params=pltpu.CompilerParams(needs_layout_passes=False),
      out_type=jax.ShapeDtypeStruct((_TOTAL // _ROW, _ROW), jnp.float32),
      scratch_types=[
          pltpu.VMEM_SHARED((_RES * _RES,), jnp.float32),
          pltpu.VMEM((2 * _PPW,), jnp.float32),
          pltpu.VMEM((_ROWS_PW, _ROW), jnp.int32),
          pltpu.VMEM((_ROWS_PW, _ROW), jnp.float32),
          pltpu.SemaphoreType.DMA,
          pltpu.SemaphoreType.DMA,
      ],
  )(_sc_kernel)
  out = run(x.reshape(-1), w.reshape(-1))
  return out.reshape(_B, _N, 1)


# speedup vs baseline: 2.5180x; 2.5180x over previous
import functools

import jax
import jax.numpy as jnp
from jax import lax
from jax.experimental import pallas as pl
from jax.experimental.pallas import tpu as pltpu
from jax.experimental.pallas import tpu_sc as plsc

_RES = 1024
_MIN_B = -2000.0
_MAX_B = 3000.0
_SCALE = (_RES - 1) / (_MAX_B - _MIN_B)

_MAGIC = 12582912.0
_CLO = _MAGIC
_CHI = _MAGIC + (_RES - 1)

_NC = 2
_NS = 16
_NW = _NC * _NS
_B, _N = 64, 4096
_TOTAL = _B * _N
_PPW = _TOTAL // _NW
_ROW = 128
_ROWS_PW = _PPW // _ROW
_VPR = _ROW // 16
_WSLICE = _RES * _RES // _NS


def _quant(v):
  t = (v + jnp.float32(-_MIN_B)) * jnp.float32(_SCALE)
  z = t + jnp.float32(_MAGIC)
  z = jnp.minimum(jnp.maximum(z, jnp.float32(_CLO)), jnp.float32(_CHI))
  return plsc.bitcast(z, jnp.int32) & (_RES - 1)


def _sc_kernel(x_hbm, w_hbm, out_hbm, wsh, xv, idxv, outv, sem, wsem):
  sid = lax.axis_index("s")
  wid = sid * _NC + lax.axis_index("c")
  pltpu.make_async_copy(
      w_hbm.at[pl.ds(sid * _WSLICE, _WSLICE)],
      wsh.at[pl.ds(sid * _WSLICE, _WSLICE)],
      wsem,
  ).start()
  pltpu.sync_copy(x_hbm.at[pl.ds(wid * (2 * _ROWS_PW), 2 * _ROWS_PW)], xv)

  def row(j, carry):
    for t in range(_VPR):
      b = xv[2 * j, pl.ds(t * 16, 16)]
      d = xv[2 * j + 1, pl.ds(t * 16, 16)]
      flat = (_quant(b) << 10) | _quant(d)
      idxv[j, pl.ds(t * 16, 16)] = flat
    return carry

  lax.fori_loop(0, _ROWS_PW, row, 0)

  pltpu.make_async_copy(
      w_hbm.at[pl.ds(sid * _WSLICE, _WSLICE)],
      wsh.at[pl.ds(sid * _WSLICE, _WSLICE)],
      wsem,
  ).wait()
  plsc.subcore_barrier()

  def fire(j, carry):
    pltpu.make_async_copy(wsh.at[idxv.at[j]], outv.at[j], sem).start()
    return carry

  lax.fori_loop(0, _ROWS_PW, fire, 0)

  def drain(j, carry):
    pltpu.make_async_copy(wsh.at[idxv.at[j]], outv.at[j], sem).wait()
    return carry

  lax.fori_loop(0, _ROWS_PW, drain, 0)
  pltpu.sync_copy(outv, out_hbm.at[pl.ds(wid * _ROWS_PW, _ROWS_PW)])


@jax.jit
def kernel(x, w):
  mesh = plsc.VectorSubcoreMesh(core_axis_name="c", subcore_axis_name="s")
  run = functools.partial(
      pl.kernel,
      mesh=mesh,
      compiler_params=pltpu.CompilerParams(needs_layout_passes=False),
      out_type=jax.ShapeDtypeStruct((_TOTAL // _ROW, _ROW), jnp.float32),
      scratch_types=[
          pltpu.VMEM_SHARED((_RES * _RES,), jnp.float32),
          pltpu.VMEM((2 * _ROWS_PW, _ROW), jnp.float32),
          pltpu.VMEM((_ROWS_PW, _ROW), jnp.int32),
          pltpu.VMEM((_ROWS_PW, _ROW), jnp.float32),
          pltpu.SemaphoreType.DMA,
          pltpu.SemaphoreType.DMA,
      ],
  )(_sc_kernel)
  x_lin = (x.transpose(0, 2, 1).reshape(_B, 2, _N // _ROW, _ROW)
           .transpose(0, 2, 1, 3).reshape(2 * _TOTAL // _ROW, _ROW))
  out = run(x_lin, w.reshape(-1))
  return out.reshape(_B, _N, 1)


# speedup vs baseline: 2.5536x; 1.0142x over previous
import functools

import jax
import jax.numpy as jnp
from jax import lax
from jax.experimental import pallas as pl
from jax.experimental.pallas import tpu as pltpu
from jax.experimental.pallas import tpu_sc as plsc

_RES = 1024
_MIN_B = -2000.0
_MAX_B = 3000.0
_SCALE = (_RES - 1) / (_MAX_B - _MIN_B)

_MAGIC = 12582912.0
_CLO = _MAGIC
_CHI = _MAGIC + (_RES - 1)

_NC = 2
_NS = 16
_NW = _NC * _NS
_B, _N = 64, 4096
_TOTAL = _B * _N
_PPW = _TOTAL // _NW
_ROW = 128
_ROWS_PW = _PPW // _ROW
_VPR = _ROW // 16
_WSLICE = _RES * _RES // _NS


def _quant(v):
  t = (v + jnp.float32(-_MIN_B)) * jnp.float32(_SCALE)
  z = t + jnp.float32(_MAGIC)
  z = jnp.minimum(jnp.maximum(z, jnp.float32(_CLO)), jnp.float32(_CHI))
  return plsc.bitcast(z, jnp.int32) & (_RES - 1)


def _sc_kernel(x_hbm, w_hbm, out_hbm, wsh, xv, idxv, outv, sem, wsem):
  sid = lax.axis_index("s")
  wid = sid * _NC + lax.axis_index("c")
  pltpu.make_async_copy(
      w_hbm.at[pl.ds(sid * _WSLICE, _WSLICE)],
      wsh.at[pl.ds(sid * _WSLICE, _WSLICE)],
      wsem,
  ).start()
  pltpu.sync_copy(x_hbm.at[pl.ds(wid * (2 * _ROWS_PW), 2 * _ROWS_PW)], xv)
  pltpu.make_async_copy(
      w_hbm.at[pl.ds(sid * _WSLICE, _WSLICE)],
      wsh.at[pl.ds(sid * _WSLICE, _WSLICE)],
      wsem,
  ).wait()
  plsc.subcore_barrier()

  def row(j, carry):
    for t in range(_VPR):
      b = xv[2 * j, pl.ds(t * 16, 16)]
      d = xv[2 * j + 1, pl.ds(t * 16, 16)]
      flat = (_quant(b) << 10) | _quant(d)
      idxv[j, pl.ds(t * 16, 16)] = flat
    pltpu.make_async_copy(wsh.at[idxv.at[j]], outv.at[j], sem).start()
    return carry

  lax.fori_loop(0, _ROWS_PW, row, 0)

  def drain(j, carry):
    pltpu.make_async_copy(wsh.at[idxv.at[j]], outv.at[j], sem).wait()
    return carry

  lax.fori_loop(0, _ROWS_PW, drain, 0)
  pltpu.sync_copy(outv, out_hbm.at[pl.ds(wid * _ROWS_PW, _ROWS_PW)])


@jax.jit
def kernel(x, w):
  mesh = plsc.VectorSubcoreMesh(core_axis_name="c", subcore_axis_name="s")
  run = functools.partial(
      pl.kernel,
      mesh=mesh,
      compiler_params=pltpu.CompilerParams(needs_layout_passes=False),
      out_type=jax.ShapeDtypeStruct((_TOTAL // _ROW, _ROW), jnp.float32),
      scratch_types=[
          pltpu.VMEM_SHARED((_RES * _RES,), jnp.float32),
          pltpu.VMEM((2 * _ROWS_PW, _ROW), jnp.float32),
          pltpu.VMEM((_ROWS_PW, _ROW), jnp.int32),
          pltpu.VMEM((_ROWS_PW, _ROW), jnp.float32),
          pltpu.SemaphoreType.DMA,
          pltpu.SemaphoreType.DMA,
      ],
  )(_sc_kernel)
  x_lin = (x.transpose(0, 2, 1).reshape(_B, 2, _N // _ROW, _ROW)
           .transpose(0, 2, 1, 3).reshape(2 * _TOTAL // _ROW, _ROW))
  out = run(x_lin, w.reshape(-1))
  return out.reshape(_B, _N, 1)


# speedup vs baseline: 2.7480x; 1.0761x over previous
import functools

import jax
import jax.numpy as jnp
from jax import lax
from jax.experimental import pallas as pl
from jax.experimental.pallas import tpu as pltpu
from jax.experimental.pallas import tpu_sc as plsc

_RES = 1024
_MIN_B = -2000.0
_MAX_B = 3000.0
_SCALE = (_RES - 1) / (_MAX_B - _MIN_B)

_MAGIC = 12582912.0
_CLO = _MAGIC
_CHI = _MAGIC + (_RES - 1)

_NC = 2
_NS = 16
_NW = _NC * _NS
_B, _N = 64, 4096
_TOTAL = _B * _N
_PPW = _TOTAL // _NW
_ROW = 128
_ROWS_PW = _PPW // _ROW
_VPR = _ROW // 16
_WSLICE = _RES * _RES // _NS


def _quant(v):
  t = (v + jnp.float32(-_MIN_B)) * jnp.float32(_SCALE)
  z = t + jnp.float32(_MAGIC)
  z = jnp.minimum(jnp.maximum(z, jnp.float32(_CLO)), jnp.float32(_CHI))
  return plsc.bitcast(z, jnp.int32)


def _sc_kernel(x_hbm, w_hbm, out_hbm, wsh, xv, idxv, outv, sem, wsem):
  sid = lax.axis_index("s")
  wid = sid * _NC + lax.axis_index("c")
  pltpu.make_async_copy(
      w_hbm.at[pl.ds(sid * _WSLICE, _WSLICE)],
      wsh.at[pl.ds(sid * _WSLICE, _WSLICE)],
      wsem,
  ).start()
  pltpu.sync_copy(x_hbm.at[pl.ds(wid * (2 * _ROWS_PW), 2 * _ROWS_PW)], xv)
  pltpu.make_async_copy(
      w_hbm.at[pl.ds(sid * _WSLICE, _WSLICE)],
      wsh.at[pl.ds(sid * _WSLICE, _WSLICE)],
      wsem,
  ).wait()
  plsc.subcore_barrier()

  def row(j, carry):
    for t in range(_VPR):
      b = xv[2 * j, pl.ds(t * 16, 16)]
      d = xv[2 * j + 1, pl.ds(t * 16, 16)]
      qb = _quant(b)
      qd = _quant(d)
      flat = (((qb & 0x3F8) << 10) | ((qb & 7) << 7)
              | ((qd & 0x380) << 3) | (qd & 127))
      idxv[j, pl.ds(t * 16, 16)] = flat
    pltpu.make_async_copy(wsh.at[idxv.at[j]], outv.at[j], sem).start()
    return carry

  lax.fori_loop(0, _ROWS_PW, row, 0)

  def drain(j, carry):
    pltpu.make_async_copy(wsh.at[idxv.at[j]], outv.at[j], sem).wait()
    return carry

  lax.fori_loop(0, _ROWS_PW, drain, 0)
  pltpu.sync_copy(outv, out_hbm.at[pl.ds(wid * _ROWS_PW, _ROWS_PW)])


@jax.jit
def kernel(x, w):
  mesh = plsc.VectorSubcoreMesh(core_axis_name="c", subcore_axis_name="s")
  run = functools.partial(
      pl.kernel,
      mesh=mesh,
      compiler_params=pltpu.CompilerParams(needs_layout_passes=False),
      out_type=jax.ShapeDtypeStruct((_TOTAL // _ROW, _ROW), jnp.float32),
      scratch_types=[
          pltpu.VMEM_SHARED((_RES * _RES,), jnp.float32),
          pltpu.VMEM((2 * _ROWS_PW, _ROW), jnp.float32),
          pltpu.VMEM((_ROWS_PW, _ROW), jnp.int32),
          pltpu.VMEM((_ROWS_PW, _ROW), jnp.float32),
          pltpu.SemaphoreType.DMA,
          pltpu.SemaphoreType.DMA,
      ],
  )(_sc_kernel)
  x_lin = (x.transpose(0, 2, 1).reshape(_B, 2, _N // _ROW, _ROW)
           .transpose(0, 2, 1, 3).reshape(2 * _TOTAL // _ROW, _ROW))
  w_lin = w.reshape(128, 8, 8, 128).transpose(0, 2, 1, 3).reshape(-1)
  out = run(x_lin, w_lin)
  return out.reshape(_B, _N, 1)


# speedup vs baseline: 6.2218x; 2.2642x over previous
import functools

import jax
import jax.numpy as jnp
from jax import lax
from jax.experimental import pallas as pl
from jax.experimental.pallas import tpu as pltpu
from jax.experimental.pallas import tpu_sc as plsc

_RES = 1024
_MIN_B = -2000.0
_MAX_B = 3000.0
_SCALE = (_RES - 1) / (_MAX_B - _MIN_B)

_MAGIC = 12582912.0
_CLO = _MAGIC
_CHI = _MAGIC + (_RES - 1)

_NC = 2
_NS = 16
_NW = _NC * _NS
_B, _N = 64, 4096
_TOTAL = _B * _N
_PPW = _TOTAL // _NW
_ROW = 128
_ROWS_PW = _PPW // _ROW
_VPR = _ROW // 16
_WSLICE = _RES * _RES // _NS

_HOT_TR0 = 48
_HOT_BLKS = 16


def _quant(v):
  t = (v + jnp.float32(-_MIN_B)) * jnp.float32(_SCALE)
  z = t + jnp.float32(_MAGIC)
  z = jnp.minimum(jnp.maximum(z, jnp.float32(_CLO)), jnp.float32(_CHI))
  return plsc.bitcast(z, jnp.int32)


def _sc_kernel(x_hbm, w_hbm, out_hbm, wsh, xv, idxv, outv, hot, rows_smem,
               sem, wsem, hsem):
  sid = lax.axis_index("s")
  wid = sid * _NC + lax.axis_index("c")
  pltpu.make_async_copy(
      w_hbm.at[pl.ds(sid * _WSLICE, _WSLICE)],
      wsh.at[pl.ds(sid * _WSLICE, _WSLICE)],
      wsem,
  ).start()
  for i in range(_HOT_BLKS):
    pltpu.make_async_copy(
        w_hbm.at[pl.ds(((_HOT_TR0 + i) * 8 + 3) * 1024, 1024)],
        hot.at[pl.ds(i * 1024, 1024)],
        hsem,
    ).start()
  pltpu.sync_copy(x_hbm.at[pl.ds(wid * (2 * _ROWS_PW), 2 * _ROWS_PW)], xv)
  for i in range(_HOT_BLKS):
    pltpu.make_async_copy(
        w_hbm.at[pl.ds(((_HOT_TR0 + i) * 8 + 3) * 1024, 1024)],
        hot.at[pl.ds(i * 1024, 1024)],
        hsem,
    ).wait()

  def row(j, n):
    coldv = None
    for t in range(_VPR):
      b = xv[2 * j, pl.ds(t * 16, 16)]
      d = xv[2 * j + 1, pl.ds(t * 16, 16)]
      qb = _quant(b)
      qd = _quant(d)
      flat = (((qb & 0x3F8) << 10) | ((qb & 7) << 7)
              | ((qd & 0x380) << 3) | (qd & 127))
      idxv[j, pl.ds(t * 16, 16)] = flat
      local = ((qb & 127) << 7) | (qd & 127)
      outv[j, pl.ds(t * 16, 16)] = plsc.load_gather(hot, [local])
      isc = (((qb >> 7) & 7) != 3) | (((qd >> 7) & 7) != 3)
      coldv = isc if coldv is None else (coldv | isc)
    ncold = plsc.all_reduce_population_count(coldv)
    c0 = jnp.max(ncold)
    rows_smem[n] = j
    return n + (c0 > 0).astype(jnp.int32)

  ncold_rows = lax.fori_loop(0, _ROWS_PW, row, 0)

  pltpu.make_async_copy(
      w_hbm.at[pl.ds(sid * _WSLICE, _WSLICE)],
      wsh.at[pl.ds(sid * _WSLICE, _WSLICE)],
      wsem,
  ).wait()
  plsc.subcore_barrier()

  def fire(k, carry):
    j = rows_smem[k]
    pltpu.make_async_copy(wsh.at[idxv.at[j]], outv.at[j], sem).start()
    return carry

  lax.fori_loop(0, ncold_rows, fire, 0)

  def drain(k, carry):
    j = rows_smem[k]
    pltpu.make_async_copy(wsh.at[idxv.at[j]], outv.at[j], sem).wait()
    return carry

  lax.fori_loop(0, ncold_rows, drain, 0)
  pltpu.sync_copy(outv, out_hbm.at[pl.ds(wid * _ROWS_PW, _ROWS_PW)])


@jax.jit
def kernel(x, w):
  mesh = plsc.VectorSubcoreMesh(core_axis_name="c", subcore_axis_name="s")
  run = functools.partial(
      pl.kernel,
      mesh=mesh,
      compiler_params=pltpu.CompilerParams(needs_layout_passes=False),
      out_type=jax.ShapeDtypeStruct((_TOTAL // _ROW, _ROW), jnp.float32),
      scratch_types=[
          pltpu.VMEM_SHARED((_RES * _RES,), jnp.float32),
          pltpu.VMEM((2 * _ROWS_PW, _ROW), jnp.float32),
          pltpu.VMEM((_ROWS_PW, _ROW), jnp.int32),
          pltpu.VMEM((_ROWS_PW, _ROW), jnp.float32),
          pltpu.VMEM((_HOT_BLKS * 1024,), jnp.float32),
          pltpu.SMEM((_ROWS_PW,), jnp.int32),
          pltpu.SemaphoreType.DMA,
          pltpu.SemaphoreType.DMA,
          pltpu.SemaphoreType.DMA,
      ],
  )(_sc_kernel)
  x_lin = (x.transpose(0, 2, 1).reshape(_B, 2, _N // _ROW, _ROW)
           .transpose(0, 2, 1, 3).reshape(2 * _TOTAL // _ROW, _ROW))
  w_lin = w.reshape(128, 8, 8, 128).transpose(0, 2, 1, 3).reshape(-1)
  out = run(x_lin, w_lin)
  return out.reshape(_B, _N, 1)
